# Initial kernel scaffold; baseline (speedup 1.0000x reference)
import jax
import jax.numpy as jnp
from jax.experimental import pallas as pl


def kernel(h, edge_index, edge_weight, params):
    raise NotImplementedError("write your pallas kernel here")


# speedup vs baseline: 1.0839x; 1.0839x over previous
import functools

import jax
import jax.numpy as jnp
from jax.experimental import pallas as pl
from jax.experimental.pallas import tpu as pltpu


def _mm_kernel(x_ref, w_ref, b_ref, o_ref):
    o_ref[...] = (
        jnp.dot(x_ref[...], w_ref[...], preferred_element_type=jnp.float32)
        + b_ref[...]
    )


def _project(x, W, b, bn=80, bo=512):
    N, D = x.shape
    O = W.shape[0]
    if D < 128:
        pad = 128 - D
        x = jnp.pad(x, ((0, 0), (0, pad)))
        W = jnp.pad(W, ((0, 0), (0, pad)))
        D = 128
    Wt = W.T
    grid = (N // bn, O // bo)
    return pl.pallas_call(
        _mm_kernel,
        grid=grid,
        in_specs=[
            pl.BlockSpec((bn, D), lambda i, j: (i, 0)),
            pl.BlockSpec((D, bo), lambda i, j: (0, j)),
            pl.BlockSpec((1, bo), lambda i, j: (0, j)),
        ],
        out_specs=pl.BlockSpec((bn, bo), lambda i, j: (i, j)),
        out_shape=jax.ShapeDtypeStruct((N, O), jnp.float32),
    )(x, Wt, b.reshape(1, O))


def _gat(x, p, src_f, dst_f, ea_f, num_nodes, H, C):
    xl = _project(x, p["Wl"], p["bl"])
    xr = _project(x, p["Wr"], p["br"])
    we_flat = p["We"].reshape(-1)
    att_flat = p["att"].reshape(-1)

    m = xl[src_f] + xr[dst_f] + ea_f * we_flat[None, :]
    m = jnp.where(m > 0, m, 0.2 * m)
    alpha = (m * att_flat[None, :]).reshape(-1, H, C).sum(-1)

    amax = jax.ops.segment_max(alpha, dst_f, num_segments=num_nodes)
    amax = jnp.where(jnp.isfinite(amax), amax, 0.0)
    ex = jnp.exp(alpha - amax[dst_f])
    den = jax.ops.segment_sum(ex, dst_f, num_segments=num_nodes)
    a = ex / (den[dst_f] + 1e-16)

    w = jnp.repeat(a, C, axis=1)
    out = jax.ops.segment_sum(xl[src_f] * w, dst_f, num_segments=num_nodes)
    out = out.reshape(num_nodes, H, C).mean(axis=1) + p["bias"]
    return out


def _bn(x, g, b):
    m = x.mean(axis=0)
    v = x.var(axis=0)
    return g * (x - m) / jnp.sqrt(v + 1e-5) + b


def kernel(h, edge_index, edge_weight, params):
    num_nodes = h.shape[0]
    H = params["conv1"]["att"].shape[0]
    C = params["conv1"]["att"].shape[1]
    src = edge_index[0]
    dst = edge_index[1]
    loop = jnp.arange(num_nodes, dtype=src.dtype)
    src_f = jnp.concatenate([src, loop])
    dst_f = jnp.concatenate([dst, loop])
    ea_mean = jnp.mean(edge_weight, axis=0, keepdims=True)
    ea_f = jnp.concatenate(
        [edge_weight, jnp.broadcast_to(ea_mean, (num_nodes, edge_weight.shape[1]))],
        axis=0,
    )

    x = _bn(h, params["bn0_g"], params["bn0_b"])
    x = jax.nn.relu(
        _bn(_gat(x, params["conv1"], src_f, dst_f, ea_f, num_nodes, H, C),
            params["bn1_g"], params["bn1_b"]))
    x = jax.nn.relu(
        _bn(_gat(x, params["conv2"], src_f, dst_f, ea_f, num_nodes, H, C),
            params["bn2_g"], params["bn2_b"]))
    mu = _gat(x, params["mu"], src_f, dst_f, ea_f, num_nodes, H, C)
    log_std = _gat(x, params["log_std"], src_f, dst_f, ea_f, num_nodes, H, C)
    return (mu, log_std)
